# Initial kernel scaffold; baseline (speedup 1.0000x reference)
import jax
import jax.numpy as jnp
from jax.experimental import pallas as pl


def kernel(x, edge_index, W1, b1, W3, b3):
    raise NotImplementedError("write your pallas kernel here")


# speedup vs baseline: 20.7894x; 20.7894x over previous
import functools

import jax
import jax.numpy as jnp
from jax import lax
from jax.experimental import pallas as pl
from jax.experimental.pallas import tpu as pltpu
from jax.experimental.pallas import tpu_sc as plsc

N = 10000
E = 320000
NC = 2
NS = 16
NW = NC * NS
E_CORE = E // NC
E_TILE = E // NW
R_TILE = 624
R_LAST = N - (NS - 1) * R_TILE
DEG_B = 2000

_mesh = plsc.VectorSubcoreMesh(core_axis_name="c", subcore_axis_name="s")


def _sc_deg(dst):
    @functools.partial(
        pl.kernel,
        out_type=jax.ShapeDtypeStruct((NC * N,), jnp.float32),
        mesh=_mesh,
        scratch_types=[
            pltpu.VMEM((DEG_B,), jnp.int32),
            pltpu.VMEM((DEG_B,), jnp.float32),
            pltpu.VMEM((1008,), jnp.float32),
            pltpu.VMEM_SHARED((N,), jnp.float32),
        ],
    )
    def k(dst_hbm, out_hbm, idx_v, ones_v, vb, acc):
        cid = lax.axis_index("c")
        sid = lax.axis_index("s")

        def fill(i, carry):
            ones_v[pl.ds(i * 16, 16)] = jnp.full((16,), 1.0, jnp.float32)
            return carry

        lax.fori_loop(0, DEG_B // 16, fill, 0)

        def zfill(i, carry):
            vb[pl.ds(i * 16, 16)] = jnp.zeros((16,), jnp.float32)
            return carry

        lax.fori_loop(0, 1008 // 16, zfill, 0)

        @pl.when(sid < 10)
        def _():
            sl = pl.ds(pl.multiple_of(sid * 1000, 8), 1000)
            pltpu.sync_copy(vb.at[pl.ds(0, 1000)], acc.at[sl])

        plsc.subcore_barrier()
        base = cid * E_CORE + sid * E_TILE
        for b in range(E_TILE // DEG_B):
            pltpu.sync_copy(dst_hbm.at[pl.ds(base + b * DEG_B, DEG_B)], idx_v)
            pltpu.sync_copy(ones_v, acc.at[idx_v], add=True)
        plsc.subcore_barrier()

        @pl.when(sid < 10)
        def _():
            sl = pl.ds(pl.multiple_of(sid * 1000, 8), 1000)
            pltpu.sync_copy(acc.at[sl], vb.at[pl.ds(0, 1000)])
            pltpu.sync_copy(vb.at[pl.ds(0, 1000)],
                            out_hbm.at[pl.ds(cid * N + sid * 1000, 1000)])

    return k(dst)


def _chunks(total, step):
    out, off = [], 0
    while off < total:
        sz = min(step, total - off)
        out.append((off, sz))
        off += sz
    return out


def _sc_segsum(src, dst, table, F, B):
    NB = E_TILE // B

    @functools.partial(
        pl.kernel,
        out_type=jax.ShapeDtypeStruct((NC, N, F), jnp.float32),
        mesh=_mesh,
        scratch_types=[
            pltpu.VMEM((B,), jnp.int32),
            pltpu.VMEM((B,), jnp.int32),
            pltpu.VMEM((B, F), jnp.float32),
            pltpu.VMEM_SHARED((N, F), jnp.float32),
            pltpu.SemaphoreType.DMA,
        ],
    )
    def k(src_hbm, dst_hbm, tab_hbm, out_hbm, sidx, didx, rows, acc, sem):
        cid = lax.axis_index("c")
        sid = lax.axis_index("s")
        r0 = pl.multiple_of(sid * R_TILE, 8)

        def zfill(i, carry):
            def zrow(j, carry2):
                rows[i, pl.ds(j * 16, 16)] = jnp.zeros((16,), jnp.float32)
                return carry2

            return lax.fori_loop(0, F // 16, zrow, carry)

        lax.fori_loop(0, B, zfill, 0)

        @pl.when(sid < NS - 1)
        def _():
            for off, sz in _chunks(R_TILE, B):
                pltpu.sync_copy(rows.at[pl.ds(0, sz), :],
                                acc.at[pl.ds(r0 + off, sz), :])

        @pl.when(sid == NS - 1)
        def _():
            for off, sz in _chunks(R_LAST, B):
                pltpu.sync_copy(rows.at[pl.ds(0, sz), :],
                                acc.at[pl.ds(r0 + off, sz), :])

        plsc.subcore_barrier()
        base = cid * E_CORE + sid * E_TILE
        for b in range(NB):
            off = base + b * B
            pltpu.sync_copy(src_hbm.at[pl.ds(off, B)], sidx)
            pltpu.sync_copy(dst_hbm.at[pl.ds(off, B)], didx)
            pltpu.async_copy(tab_hbm.at[sidx], rows, sem).wait()
            pltpu.sync_copy(rows, acc.at[didx], add=True)
        plsc.subcore_barrier()

        @pl.when(sid < NS - 1)
        def _():
            for off, sz in _chunks(R_TILE, B):
                pltpu.sync_copy(acc.at[pl.ds(r0 + off, sz), :],
                                rows.at[pl.ds(0, sz), :])
                pltpu.sync_copy(rows.at[pl.ds(0, sz), :],
                                out_hbm.at[cid, pl.ds(r0 + off, sz), :])

        @pl.when(sid == NS - 1)
        def _():
            for off, sz in _chunks(R_LAST, B):
                pltpu.sync_copy(acc.at[pl.ds(r0 + off, sz), :],
                                rows.at[pl.ds(0, sz), :])
                pltpu.sync_copy(rows.at[pl.ds(0, sz), :],
                                out_hbm.at[cid, pl.ds(r0 + off, sz), :])

    return k(src, dst, table)


def _tc1(x, degp3):
    def body(x_ref, d_ref, o_ref):
        dinv = lax.rsqrt(d_ref[0] + d_ref[1] + 1.0)
        o_ref[...] = x_ref[...] * dinv

    return pl.pallas_call(
        body, out_shape=jax.ShapeDtypeStruct((N, 128), jnp.float32)
    )(x, degp3)


def _tc2(s1, xt, degp3, W1, b1, W3):
    def body(s_ref, x_ref, d_ref, w1_ref, b_ref, w3_ref, o_ref):
        dinv = lax.rsqrt(d_ref[0] + d_ref[1] + 1.0)
        agg = (s_ref[0] + s_ref[1] + x_ref[...]) * dinv
        h = jnp.dot(agg, w1_ref[...], preferred_element_type=jnp.float32)
        h = jnp.maximum(h + b_ref[...], 0.0)
        o_ref[...] = jnp.dot(h, w3_ref[...], preferred_element_type=jnp.float32) * dinv

    return pl.pallas_call(
        body, out_shape=jax.ShapeDtypeStruct((N, W3.shape[1]), jnp.float32)
    )(s1, xt, degp3, W1, b1, W3)


def _tc3(s2, g2, degp3, b3):
    def body(s_ref, g_ref, d_ref, b_ref, o_ref):
        dinv = lax.rsqrt(d_ref[0] + d_ref[1] + 1.0)
        z = (s_ref[0] + s_ref[1] + g_ref[...]) * dinv + b_ref[...]
        o_ref[...] = jnp.maximum(z, 0.0)

    return pl.pallas_call(
        body, out_shape=jax.ShapeDtypeStruct((N, g2.shape[1]), jnp.float32)
    )(s2, g2, degp3, b3)


def kernel(x, edge_index, W1, b1, W3, b3):
    ei = edge_index.astype(jnp.int32)
    src, dst = ei[0], ei[1]
    degp = _sc_deg(dst)
    degp3 = degp.reshape(NC, N, 1)
    xt = _tc1(x, degp3)
    s1 = _sc_segsum(src, dst, xt, 128, 200)
    g2 = _tc2(s1, xt, degp3, W1, b1, W3)
    s2 = _sc_segsum(src, dst, g2, 128, 200)
    return _tc3(s2, g2, degp3, b3)
